# baseline (speedup 1.0000x reference)
import functools

import jax
import jax.numpy as jnp
from jax import lax
from jax.experimental import pallas as pl
from jax.experimental.pallas import tpu as pltpu, tpu_sc as plsc

BATCH = 16384
DIM = 32
NW = 32
B_PER_W = BATCH // NW
CHUNK = 128
NCH = B_PER_W // CHUNK


def _body(uidx_hbm, sidx_hbm, utab_hbm, itab_hbm, out_hbm,
          uidx_v, sidx_v, urows_v, irows_v, out_v, sem):
    wid = lax.axis_index("s") * 2 + lax.axis_index("c")
    base = wid * NCH

    pltpu.sync_copy(uidx_hbm.at[pl.ds(base, NCH)], uidx_v)
    pltpu.sync_copy(sidx_hbm.at[pl.ds(base, NCH)], sidx_v)

    copies = []
    for j in range(NCH):
        copies.append(pltpu.async_copy(utab_hbm.at[uidx_v.at[j]],
                                       urows_v.at[j], sem))
        copies.append(pltpu.async_copy(itab_hbm.at[sidx_v.at[j]],
                                       irows_v.at[j], sem))
    for c in copies:
        c.wait()

    lanes = lax.iota(jnp.int32, 16)
    for j in range(NCH):
        @pl.loop(0, CHUNK // 16)
        def _(g, j=j):
            res = jnp.zeros((16,), jnp.float32)
            for i in range(16):
                b = g * 16 + i
                u0 = urows_v[j, b, pl.ds(0, 16)]
                u1 = urows_v[j, b, pl.ds(16, 16)]
                v0 = irows_v[j, b, pl.ds(0, 16)]
                v1 = irows_v[j, b, pl.ds(16, 16)]
                s = jnp.sum(u0 * v0 + u1 * v1)
                res = jnp.where(lanes == i, s, res)
            out_v[j, pl.ds(g * 16, 16)] = res

    pltpu.sync_copy(out_v, out_hbm.at[pl.ds(base, NCH)])


@jax.jit
def _cf_sc(userIdx, servIdx, user_table, item_table):
    uidx = userIdx.astype(jnp.int32).reshape(NW * NCH, CHUNK)
    sidx = servIdx.astype(jnp.int32).reshape(NW * NCH, CHUNK)

    mesh = plsc.VectorSubcoreMesh(core_axis_name="c", subcore_axis_name="s")
    out = pl.kernel(
        _body,
        out_type=jax.ShapeDtypeStruct((NW * NCH, CHUNK), jnp.float32),
        mesh=mesh,
        compiler_params=pltpu.CompilerParams(
            needs_layout_passes=False, use_tc_tiling_on_sc=False),
        scratch_types=[
            pltpu.VMEM((NCH, CHUNK), jnp.int32),
            pltpu.VMEM((NCH, CHUNK), jnp.int32),
            pltpu.VMEM((NCH, CHUNK, DIM), jnp.float32),
            pltpu.VMEM((NCH, CHUNK, DIM), jnp.float32),
            pltpu.VMEM((NCH, CHUNK), jnp.float32),
            pltpu.SemaphoreType.DMA,
        ],
    )(uidx, sidx, user_table, item_table)
    return out.reshape(BATCH)


def kernel(userIdx, servIdx, user_table, item_table):
    return _cf_sc(userIdx, servIdx, user_table, item_table)
